# Initial kernel scaffold; baseline (speedup 1.0000x reference)
import functools

import jax
import jax.numpy as jnp
from jax import lax
from jax.experimental import pallas as pl
from jax.experimental.pallas import tpu as pltpu
from jax.experimental.pallas import tpu_sc as plsc

N = 50000
E = 800000
D_IN = 64
D_HID = 32
N_CLASS = 40

NC = 2
NS = 16
NW = NC * NS
CHUNK = 128
N_PAD_ROWS = 176
N_ACC = N + N_PAD_ROWS
ROWS_PER_TILE = N_ACC // NS
N_CHUNKS = -(-E // (NW * CHUNK))
E_PAD = NW * CHUNK * N_CHUNKS
EW = E_PAD // NW
ZROWS = ROWS_PER_TILE // 8

_mesh = plsc.VectorSubcoreMesh(core_axis_name="c", subcore_axis_name="s")


@functools.partial(
    pl.kernel,
    out_type=jax.ShapeDtypeStruct((NC, N_ACC), jnp.float32),
    mesh=_mesh,
    scratch_types=[
        pltpu.VMEM((N_CHUNKS, CHUNK), jnp.int32),
        pltpu.VMEM((CHUNK,), jnp.float32),
        pltpu.VMEM((ROWS_PER_TILE,), jnp.float32),
        pltpu.VMEM_SHARED((N_ACC,), jnp.float32),
    ],
)
def _deg_kernel(dst_hbm, deg_out, dst_v, ones_v, zb_v, acc_sh):
    c = lax.axis_index("c")
    s = lax.axis_index("s")
    w = s * NC + c

    def fill(i, _):
        ones_v[pl.ds(i * 16, 16)] = jnp.ones((16,), jnp.float32)
        return 0

    lax.fori_loop(0, CHUNK // 16, fill, 0)

    def zfill(i, _):
        zb_v[pl.ds(i * 16, 16)] = jnp.zeros((16,), jnp.float32)
        return 0

    lax.fori_loop(0, ROWS_PER_TILE // 16, zfill, 0)

    pltpu.sync_copy(zb_v, acc_sh.at[pl.ds(s * ROWS_PER_TILE, ROWS_PER_TILE)])
    plsc.subcore_barrier()

    pltpu.sync_copy(dst_hbm.at[w], dst_v)

    def body(j, _):
        pltpu.sync_copy(ones_v, acc_sh.at[dst_v.at[j]], add=True)
        return 0

    lax.fori_loop(0, N_CHUNKS, body, 0)
    plsc.subcore_barrier()

    pltpu.sync_copy(
        acc_sh.at[pl.ds(s * ROWS_PER_TILE, ROWS_PER_TILE)],
        deg_out.at[c, pl.ds(s * ROWS_PER_TILE, ROWS_PER_TILE)],
    )


@functools.partial(
    pl.kernel,
    out_type=jax.ShapeDtypeStruct((NC, N_ACC, D_HID), jnp.float32),
    mesh=_mesh,
    scratch_types=[
        pltpu.VMEM((N_CHUNKS, CHUNK), jnp.int32),
        pltpu.VMEM((N_CHUNKS, CHUNK), jnp.int32),
        pltpu.VMEM((2, CHUNK, D_HID), jnp.float32),
        pltpu.VMEM((ZROWS, D_HID), jnp.float32),
        pltpu.VMEM_SHARED((N_ACC, D_HID), jnp.float32),
        pltpu.SemaphoreType.DMA,
        pltpu.SemaphoreType.DMA,
    ],
)
def _scatter_kernel(y_hbm, src_hbm, dst_hbm, acc_out,
                    src_v, dst_v, rows_v, zb_v, acc_sh, sem0, sem1):
    c = lax.axis_index("c")
    s = lax.axis_index("s")
    w = s * NC + c
    sems = [sem0, sem1]

    def zfill(i, _):
        zb_v[i, pl.ds(0, 16)] = jnp.zeros((16,), jnp.float32)
        zb_v[i, pl.ds(16, 16)] = jnp.zeros((16,), jnp.float32)
        return 0

    lax.fori_loop(0, ZROWS, zfill, 0)
    for z in range(8):
        pltpu.sync_copy(
            zb_v, acc_sh.at[pl.ds(s * ROWS_PER_TILE + z * ZROWS, ZROWS)]
        )
    plsc.subcore_barrier()

    pltpu.sync_copy(src_hbm.at[w], src_v)
    pltpu.sync_copy(dst_hbm.at[w], dst_v)

    pltpu.async_copy(y_hbm.at[src_v.at[0]], rows_v.at[0], sem0)

    def body(jj, _):
        for b in range(2):
            j = 2 * jj + b
            pltpu.make_async_copy(
                y_hbm.at[src_v.at[j]], rows_v.at[b], sems[b]
            ).wait()

            @pl.when(j + 1 < N_CHUNKS)
            def _():
                pltpu.async_copy(
                    y_hbm.at[src_v.at[j + 1]], rows_v.at[1 - b], sems[1 - b]
                )

            pltpu.sync_copy(rows_v.at[b], acc_sh.at[dst_v.at[j]], add=True)
        return 0

    lax.fori_loop(0, N_CHUNKS // 2, body, 0)
    plsc.subcore_barrier()

    for z in range(8):
        r0 = s * ROWS_PER_TILE + z * ZROWS
        pltpu.sync_copy(
            acc_sh.at[pl.ds(r0, ZROWS)], acc_out.at[c, pl.ds(r0, ZROWS)]
        )


def _dense1_body(x_ref, deg_ref, w1_ref, y_ref):
    deg = deg_ref[0, :] + deg_ref[1, :] + 1.0
    dinv = lax.rsqrt(deg)
    xw = jnp.dot(x_ref[...], w1_ref[...], preferred_element_type=jnp.float32)
    y_ref[...] = xw * dinv[:, None]


def _dense2_body(acc_ref, y_ref, deg_ref, b1_ref, w2_ref, b2_ref, z_ref):
    deg = deg_ref[0, :] + deg_ref[1, :] + 1.0
    dinv = lax.rsqrt(deg)
    tot = acc_ref[0] + acc_ref[1] + y_ref[...]
    h = tot * dinv[:, None] + b1_ref[...]
    e = jnp.maximum(h, 0.0)
    z_ref[...] = (
        jnp.dot(e, w2_ref[...], preferred_element_type=jnp.float32)
        + b2_ref[...]
    )


RB = 2000


def kernel(x, edge_index, W1, b1, W2, b2):
    src = edge_index[0].astype(jnp.int32)
    dst = edge_index[1].astype(jnp.int32)
    npad = E_PAD - E
    ar = jnp.arange(npad, dtype=jnp.int32)
    src_p = jnp.concatenate([src, ar % CHUNK]).reshape(NW, N_CHUNKS, CHUNK)
    dst_p = jnp.concatenate([dst, N + (ar % N_PAD_ROWS)]).reshape(
        NW, N_CHUNKS, CHUNK
    )

    deg = _deg_kernel(dst_p)

    y = pl.pallas_call(
        _dense1_body,
        grid=(N // RB,),
        in_specs=[
            pl.BlockSpec((RB, D_IN), lambda i: (i, 0)),
            pl.BlockSpec((NC, RB), lambda i: (0, i)),
            pl.BlockSpec((D_IN, D_HID), lambda i: (0, 0)),
        ],
        out_specs=pl.BlockSpec((RB, D_HID), lambda i: (i, 0)),
        out_shape=jax.ShapeDtypeStruct((N, D_HID), jnp.float32),
    )(x, deg, W1)

    acc = _scatter_kernel(y, src_p, dst_p)

    z = pl.pallas_call(
        _dense2_body,
        grid=(N // RB,),
        in_specs=[
            pl.BlockSpec((NC, RB, D_HID), lambda i: (0, i, 0)),
            pl.BlockSpec((RB, D_HID), lambda i: (i, 0)),
            pl.BlockSpec((NC, RB), lambda i: (0, i)),
            pl.BlockSpec((1, D_HID), lambda i: (0, 0)),
            pl.BlockSpec((D_HID, N_CLASS), lambda i: (0, 0)),
            pl.BlockSpec((1, N_CLASS), lambda i: (0, 0)),
        ],
        out_specs=pl.BlockSpec((RB, N_CLASS), lambda i: (i, 0)),
        out_shape=jax.ShapeDtypeStruct((N, N_CLASS), jnp.float32),
    )(acc, y, deg, b1.reshape(1, D_HID), W2, b2.reshape(1, N_CLASS))

    return z


# speedup vs baseline: 31.3757x; 31.3757x over previous
import functools

import jax
import jax.numpy as jnp
from jax import lax
from jax.experimental import pallas as pl
from jax.experimental.pallas import tpu as pltpu
from jax.experimental.pallas import tpu_sc as plsc

N = 50000
E = 800000
D_IN = 64
D_HID = 32
D_HALF = D_HID // 2
N_CLASS = 40

NC = 2
NS = 16
NW = NC * NS
CHUNK = 128
N_PAD_ROWS = 176
N_ACC = N + N_PAD_ROWS
ROWS_PER_TILE = N_ACC // NS
ZROWS = ROWS_PER_TILE // 8

NCH_DEG = -(-E // (NW * CHUNK))
E_PAD = NW * CHUNK * NCH_DEG
NCH_SC = E_PAD // (NS * CHUNK)
IB = 56
NBLK = NCH_SC // IB

_mesh = plsc.VectorSubcoreMesh(core_axis_name="c", subcore_axis_name="s")


@functools.partial(
    pl.kernel,
    out_type=jax.ShapeDtypeStruct((NC * N_ACC,), jnp.float32),
    mesh=_mesh,
    scratch_types=[
        pltpu.VMEM((NCH_DEG, CHUNK), jnp.int32),
        pltpu.VMEM((CHUNK,), jnp.float32),
        pltpu.VMEM((ROWS_PER_TILE,), jnp.float32),
        pltpu.VMEM_SHARED((N_ACC,), jnp.float32),
    ],
)
def _deg_kernel(dst_hbm, deg_out, dst_v, ones_v, zb_v, acc_sh):
    c = lax.axis_index("c")
    s = lax.axis_index("s")
    w = s * NC + c

    def fill(i, _):
        ones_v[pl.ds(i * 16, 16)] = jnp.ones((16,), jnp.float32)
        return 0

    lax.fori_loop(0, CHUNK // 16, fill, 0)

    def zfill(i, _):
        zb_v[pl.ds(i * 16, 16)] = jnp.zeros((16,), jnp.float32)
        return 0

    lax.fori_loop(0, ROWS_PER_TILE // 16, zfill, 0)

    pltpu.sync_copy(zb_v, acc_sh.at[pl.ds(s * ROWS_PER_TILE, ROWS_PER_TILE)])
    plsc.subcore_barrier()

    pltpu.sync_copy(dst_hbm.at[w], dst_v)

    def body(j, _):
        pltpu.sync_copy(ones_v, acc_sh.at[dst_v.at[j]], add=True)
        return 0

    lax.fori_loop(0, NCH_DEG, body, 0)
    plsc.subcore_barrier()

    pltpu.sync_copy(acc_sh.at[pl.ds(s * ROWS_PER_TILE, ROWS_PER_TILE)], zb_v)
    pltpu.sync_copy(
        zb_v, deg_out.at[pl.ds(c * N_ACC + s * ROWS_PER_TILE, ROWS_PER_TILE)]
    )


@functools.partial(
    pl.kernel,
    out_type=jax.ShapeDtypeStruct((NC, N_ACC, D_HALF), jnp.float32),
    mesh=_mesh,
    scratch_types=[
        pltpu.VMEM((IB, CHUNK), jnp.int32),
        pltpu.VMEM((IB, CHUNK), jnp.int32),
        pltpu.VMEM((2, CHUNK, D_HALF), jnp.float32),
        pltpu.VMEM((ZROWS, D_HALF), jnp.float32),
        pltpu.VMEM_SHARED((N_ACC, D_HALF), jnp.float32),
        pltpu.SemaphoreType.DMA,
        pltpu.SemaphoreType.DMA,
    ],
    compiler_params=pltpu.CompilerParams(use_tc_tiling_on_sc=False),
)
def _scatter_kernel(y_hbm, src_hbm, dst_hbm, acc_out,
                    src_v, dst_v, rows_v, zb_v, acc_sh, sem0, sem1):
    c = lax.axis_index("c")
    s = lax.axis_index("s")
    sems = [sem0, sem1]

    def zfill(i, _):
        zb_v[i, pl.ds(0, 16)] = jnp.zeros((16,), jnp.float32)
        return 0

    lax.fori_loop(0, ZROWS, zfill, 0)
    for z in range(8):
        pltpu.sync_copy(
            zb_v, acc_sh.at[pl.ds(s * ROWS_PER_TILE + z * ZROWS, ZROWS)]
        )
    plsc.subcore_barrier()

    def blk_body(blk, _):
        pltpu.sync_copy(src_hbm.at[c, s, pl.ds(blk * IB, IB)], src_v)
        pltpu.sync_copy(dst_hbm.at[s, pl.ds(blk * IB, IB)], dst_v)

        pltpu.async_copy(y_hbm.at[src_v.at[0]], rows_v.at[0], sem0)

        def body(jj, _):
            for b in range(2):
                j = 2 * jj + b
                pltpu.make_async_copy(
                    y_hbm.at[src_v.at[j]], rows_v.at[b], sems[b]
                ).wait()

                @pl.when(j + 1 < IB)
                def _():
                    pltpu.async_copy(
                        y_hbm.at[src_v.at[j + 1]], rows_v.at[1 - b],
                        sems[1 - b]
                    )

                pltpu.sync_copy(rows_v.at[b], acc_sh.at[dst_v.at[j]], add=True)
            return 0

        lax.fori_loop(0, IB // 2, body, 0)
        return 0

    lax.fori_loop(0, NBLK, blk_body, 0)
    plsc.subcore_barrier()

    for z in range(8):
        r0 = s * ROWS_PER_TILE + z * ZROWS
        pltpu.sync_copy(acc_sh.at[pl.ds(r0, ZROWS)], zb_v)
        pltpu.sync_copy(zb_v, acc_out.at[c, pl.ds(r0, ZROWS)])


def _dense1_body(x_ref, deg_ref, w1_ref, y_ref):
    deg = deg_ref[:, 0] + deg_ref[:, 1] + 1.0
    dinv = lax.rsqrt(deg)
    xw = jnp.dot(x_ref[...], w1_ref[...], preferred_element_type=jnp.float32)
    y = xw * dinv[:, None]
    y_ref[0] = y[:, :D_HALF]
    y_ref[1] = y[:, D_HALF:]


def _dense2_body(acc_ref, y_ref, deg_ref, b1_ref, w2_ref, b2_ref, z_ref):
    deg = deg_ref[:, 0] + deg_ref[:, 1] + 1.0
    dinv = lax.rsqrt(deg)
    tot = jnp.concatenate(
        [acc_ref[0] + y_ref[0], acc_ref[1] + y_ref[1]], axis=1
    )
    h = tot * dinv[:, None] + b1_ref[...]
    e = jnp.maximum(h, 0.0)
    z_ref[...] = (
        jnp.dot(e, w2_ref[...], preferred_element_type=jnp.float32)
        + b2_ref[...]
    )


RB = 2000


def kernel(x, edge_index, W1, b1, W2, b2):
    src = edge_index[0].astype(jnp.int32)
    dst = edge_index[1].astype(jnp.int32)
    npad = E_PAD - E
    ar = jnp.arange(npad, dtype=jnp.int32)
    src_p = jnp.concatenate([src, ar % CHUNK])
    dst_p = jnp.concatenate([dst, N + (ar % N_PAD_ROWS)])

    deg = _deg_kernel(dst_p.reshape(NW, NCH_DEG, CHUNK))
    deg = deg.reshape(NC, N_ACC).T

    y2 = pl.pallas_call(
        _dense1_body,
        grid=(N // RB,),
        in_specs=[
            pl.BlockSpec((RB, D_IN), lambda i: (i, 0)),
            pl.BlockSpec((RB, NC), lambda i: (i, 0)),
            pl.BlockSpec((D_IN, D_HID), lambda i: (0, 0)),
        ],
        out_specs=pl.BlockSpec((NC, RB, D_HALF), lambda i: (0, i, 0)),
        out_shape=jax.ShapeDtypeStruct((NC, N, D_HALF), jnp.float32),
    )(x, deg, W1)

    src_cc = jnp.stack([src_p, src_p + N]).reshape(NC, NS, NCH_SC, CHUNK)
    acc = _scatter_kernel(
        y2.reshape(NC * N, D_HALF),
        src_cc,
        dst_p.reshape(NS, NCH_SC, CHUNK),
    )

    z = pl.pallas_call(
        _dense2_body,
        grid=(N // RB,),
        in_specs=[
            pl.BlockSpec((NC, RB, D_HALF), lambda i: (0, i, 0)),
            pl.BlockSpec((NC, RB, D_HALF), lambda i: (0, i, 0)),
            pl.BlockSpec((RB, NC), lambda i: (i, 0)),
            pl.BlockSpec((1, D_HID), lambda i: (0, 0)),
            pl.BlockSpec((D_HID, N_CLASS), lambda i: (0, 0)),
            pl.BlockSpec((1, N_CLASS), lambda i: (0, 0)),
        ],
        out_specs=pl.BlockSpec((RB, N_CLASS), lambda i: (i, 0)),
        out_shape=jax.ShapeDtypeStruct((N, N_CLASS), jnp.float32),
    )(acc, y2, deg, b1.reshape(1, D_HID), W2, b2.reshape(1, N_CLASS))

    return z


# speedup vs baseline: 41.0820x; 1.3094x over previous
import functools

import jax
import jax.numpy as jnp
from jax import lax
from jax.experimental import pallas as pl
from jax.experimental.pallas import tpu as pltpu
from jax.experimental.pallas import tpu_sc as plsc

N = 50000
E = 800000
D_IN = 64
D_HID = 32
D_HALF = D_HID // 2
N_CLASS = 40

NC = 2
NS = 16
NW = NC * NS
CHUNK = 128
N_PAD_ROWS = 176
N_ACC = N + N_PAD_ROWS
ROWS_PER_TILE = N_ACC // NS
ZROWS = ROWS_PER_TILE // 8

NCH_DEG = -(-E // (NW * CHUNK))
E_PAD = NW * CHUNK * NCH_DEG
NCH_SC = E_PAD // (NS * CHUNK)
IB = 56
NBLK = NCH_SC // IB
WROWS = ROWS_PER_TILE // 4

_mesh = plsc.VectorSubcoreMesh(core_axis_name="c", subcore_axis_name="s")


@functools.partial(
    pl.kernel,
    out_type=jax.ShapeDtypeStruct((NC * N_ACC,), jnp.float32),
    mesh=_mesh,
    scratch_types=[
        pltpu.VMEM((NCH_DEG, CHUNK), jnp.int32),
        pltpu.VMEM((CHUNK,), jnp.float32),
        pltpu.VMEM((ROWS_PER_TILE,), jnp.float32),
        pltpu.VMEM_SHARED((N_ACC,), jnp.float32),
    ],
)
def _deg_kernel(dst_hbm, deg_out, dst_v, ones_v, zb_v, acc_sh):
    c = lax.axis_index("c")
    s = lax.axis_index("s")
    w = s * NC + c

    def fill(i, _):
        ones_v[pl.ds(i * 16, 16)] = jnp.ones((16,), jnp.float32)
        return 0

    lax.fori_loop(0, CHUNK // 16, fill, 0)

    def zfill(i, _):
        zb_v[pl.ds(i * 16, 16)] = jnp.zeros((16,), jnp.float32)
        return 0

    lax.fori_loop(0, ROWS_PER_TILE // 16, zfill, 0)

    pltpu.sync_copy(zb_v, acc_sh.at[pl.ds(s * ROWS_PER_TILE, ROWS_PER_TILE)])
    plsc.subcore_barrier()

    pltpu.sync_copy(dst_hbm.at[w], dst_v)

    def body(j, _):
        pltpu.sync_copy(ones_v, acc_sh.at[dst_v.at[j]], add=True)
        return 0

    lax.fori_loop(0, NCH_DEG, body, 0)
    plsc.subcore_barrier()

    pltpu.sync_copy(acc_sh.at[pl.ds(s * ROWS_PER_TILE, ROWS_PER_TILE)], zb_v)
    pltpu.sync_copy(
        zb_v, deg_out.at[pl.ds(c * N_ACC + s * ROWS_PER_TILE, ROWS_PER_TILE)]
    )


@functools.partial(
    pl.kernel,
    out_type=jax.ShapeDtypeStruct((NC, N_ACC, D_HALF), jnp.float32),
    mesh=_mesh,
    scratch_types=[
        pltpu.VMEM((IB, CHUNK), jnp.int32),
        pltpu.VMEM((IB, CHUNK), jnp.int32),
        pltpu.VMEM((4, CHUNK, D_HALF), jnp.float32),
        pltpu.VMEM((2, WROWS, D_HALF), jnp.float32),
        pltpu.VMEM_SHARED((N_ACC, D_HALF), jnp.float32),
        pltpu.SemaphoreType.DMA,
        pltpu.SemaphoreType.DMA,
        pltpu.SemaphoreType.DMA,
    ],
    compiler_params=pltpu.CompilerParams(use_tc_tiling_on_sc=False),
)
def _scatter_kernel(y_hbm, src_hbm, dst_hbm, acc_out,
                    src_v, dst_v, rows_v, zb_v, acc_sh, gsem, ssem, wsem):
    c = lax.axis_index("c")
    s = lax.axis_index("s")

    def zfill(i, _):
        zb_v[0, i, pl.ds(0, 16)] = jnp.zeros((16,), jnp.float32)
        return 0

    lax.fori_loop(0, WROWS, zfill, 0)
    for z in range(4):
        pltpu.sync_copy(
            zb_v.at[0], acc_sh.at[pl.ds(s * ROWS_PER_TILE + z * WROWS, WROWS)]
        )
    plsc.subcore_barrier()

    def wait_gather(j, b):
        pltpu.make_async_copy(
            y_hbm.at[src_v.at[j]], rows_v.at[b], gsem
        ).wait()

    def wait_scatter():
        pltpu.make_async_copy(
            rows_v.at[0], acc_sh.at[dst_v.at[0]], ssem
        ).wait()

    def blk_body(blk, _):
        pltpu.sync_copy(src_hbm.at[c, s, pl.ds(blk * IB, IB)], src_v)
        pltpu.sync_copy(dst_hbm.at[s, pl.ds(blk * IB, IB)], dst_v)

        pltpu.async_copy(y_hbm.at[src_v.at[0]], rows_v.at[0], gsem)
        pltpu.async_copy(y_hbm.at[src_v.at[1]], rows_v.at[1], gsem)

        def body(ii, _):
            for b in range(4):
                j = 4 * ii + b
                @pl.when(j + 2 < IB)
                def _():
                    @pl.when(j >= 2)
                    def _():
                        wait_scatter()

                    pltpu.async_copy(
                        y_hbm.at[src_v.at[j + 2]], rows_v.at[(b + 2) % 4],
                        gsem,
                    )

                wait_gather(j, b)
                pltpu.async_copy(
                    rows_v.at[b], acc_sh.at[dst_v.at[j]], ssem, add=True
                )
            return 0

        lax.fori_loop(0, IB // 4, body, 0)
        for _ in range(4):
            wait_scatter()
        return 0

    lax.fori_loop(0, NBLK, blk_body, 0)
    plsc.subcore_barrier()

    def wait_writeout(z):
        r0 = s * ROWS_PER_TILE + z * WROWS
        pltpu.make_async_copy(
            zb_v.at[z % 2], acc_out.at[c, pl.ds(r0, WROWS)], wsem
        ).wait()

    for z in range(4):
        if z >= 2:
            wait_writeout(z - 2)
        r0 = s * ROWS_PER_TILE + z * WROWS
        pltpu.sync_copy(acc_sh.at[pl.ds(r0, WROWS)], zb_v.at[z % 2])
        pltpu.async_copy(zb_v.at[z % 2], acc_out.at[c, pl.ds(r0, WROWS)], wsem)
    wait_writeout(2)
    wait_writeout(3)


def _dense1_body(x_ref, deg_ref, w1_ref, y_ref):
    deg = deg_ref[:, 0] + deg_ref[:, 1] + 1.0
    dinv = lax.rsqrt(deg)
    xw = jnp.dot(x_ref[...], w1_ref[...], preferred_element_type=jnp.float32)
    y = xw * dinv[:, None]
    y_ref[0] = y[:, :D_HALF]
    y_ref[1] = y[:, D_HALF:]


def _dense2_body(acc_ref, y_ref, deg_ref, b1_ref, w2_ref, b2_ref, z_ref):
    deg = deg_ref[:, 0] + deg_ref[:, 1] + 1.0
    dinv = lax.rsqrt(deg)
    tot = jnp.concatenate(
        [acc_ref[0] + y_ref[0], acc_ref[1] + y_ref[1]], axis=1
    )
    h = tot * dinv[:, None] + b1_ref[...]
    e = jnp.maximum(h, 0.0)
    z_ref[...] = (
        jnp.dot(e, w2_ref[...], preferred_element_type=jnp.float32)
        + b2_ref[...]
    )


RB = 2000


def kernel(x, edge_index, W1, b1, W2, b2):
    src = edge_index[0].astype(jnp.int32)
    dst = edge_index[1].astype(jnp.int32)
    npad = E_PAD - E
    ar = jnp.arange(npad, dtype=jnp.int32)
    src_p = jnp.concatenate([src, ar % CHUNK])
    dst_p = jnp.concatenate([dst, N + (ar % N_PAD_ROWS)])

    deg = _deg_kernel(dst_p.reshape(NW, NCH_DEG, CHUNK))
    deg = deg.reshape(NC, N_ACC).T

    y2 = pl.pallas_call(
        _dense1_body,
        grid=(N // RB,),
        in_specs=[
            pl.BlockSpec((RB, D_IN), lambda i: (i, 0)),
            pl.BlockSpec((RB, NC), lambda i: (i, 0)),
            pl.BlockSpec((D_IN, D_HID), lambda i: (0, 0)),
        ],
        out_specs=pl.BlockSpec((NC, RB, D_HALF), lambda i: (0, i, 0)),
        out_shape=jax.ShapeDtypeStruct((NC, N, D_HALF), jnp.float32),
    )(x, deg, W1)

    src_cc = jnp.stack([src_p, src_p + N]).reshape(NC, NS, NCH_SC, CHUNK)
    acc = _scatter_kernel(
        y2.reshape(NC * N, D_HALF),
        src_cc,
        dst_p.reshape(NS, NCH_SC, CHUNK),
    )

    z = pl.pallas_call(
        _dense2_body,
        grid=(N // RB,),
        in_specs=[
            pl.BlockSpec((NC, RB, D_HALF), lambda i: (0, i, 0)),
            pl.BlockSpec((NC, RB, D_HALF), lambda i: (0, i, 0)),
            pl.BlockSpec((RB, NC), lambda i: (i, 0)),
            pl.BlockSpec((1, D_HID), lambda i: (0, 0)),
            pl.BlockSpec((D_HID, N_CLASS), lambda i: (0, 0)),
            pl.BlockSpec((1, N_CLASS), lambda i: (0, 0)),
        ],
        out_specs=pl.BlockSpec((RB, N_CLASS), lambda i: (i, 0)),
        out_shape=jax.ShapeDtypeStruct((N, N_CLASS), jnp.float32),
    )(acc, y2, deg, b1.reshape(1, D_HID), W2, b2.reshape(1, N_CLASS))

    return z


# speedup vs baseline: 58.0960x; 1.4141x over previous
import functools

import jax
import jax.numpy as jnp
from jax import lax
from jax.experimental import pallas as pl
from jax.experimental.pallas import tpu as pltpu
from jax.experimental.pallas import tpu_sc as plsc

N = 50000
E = 800000
D_IN = 64
D_HID = 32
N_CLASS = 40

NC = 2
NS = 16
NW = NC * NS
CHUNK = 128
N_PAD_ROWS = 176
N_ACC = N + N_PAD_ROWS
ROWS_PER_TILE = N_ACC // NS
ZROWS = ROWS_PER_TILE // 8

NCH_DEG = -(-E // (NW * CHUNK))
E_PAD = NW * CHUNK * NCH_DEG
NCH_SC = E_PAD // (NC * NS * CHUNK)
IB = 28
NBLK = NCH_SC // IB
WROWS = ROWS_PER_TILE // 32

_mesh = plsc.VectorSubcoreMesh(core_axis_name="c", subcore_axis_name="s")


@functools.partial(
    pl.kernel,
    out_type=jax.ShapeDtypeStruct((NC * N_ACC,), jnp.float32),
    mesh=_mesh,
    scratch_types=[
        pltpu.VMEM((NCH_DEG, CHUNK), jnp.int32),
        pltpu.VMEM((CHUNK,), jnp.float32),
        pltpu.VMEM((ROWS_PER_TILE,), jnp.float32),
        pltpu.VMEM_SHARED((N_ACC,), jnp.float32),
    ],
)
def _deg_kernel(dst_hbm, deg_out, dst_v, ones_v, zb_v, acc_sh):
    c = lax.axis_index("c")
    s = lax.axis_index("s")
    w = s * NC + c

    def fill(i, _):
        ones_v[pl.ds(i * 16, 16)] = jnp.ones((16,), jnp.float32)
        return 0

    lax.fori_loop(0, CHUNK // 16, fill, 0)

    def zfill(i, _):
        zb_v[pl.ds(i * 16, 16)] = jnp.zeros((16,), jnp.float32)
        return 0

    lax.fori_loop(0, ROWS_PER_TILE // 16, zfill, 0)

    pltpu.sync_copy(zb_v, acc_sh.at[pl.ds(s * ROWS_PER_TILE, ROWS_PER_TILE)])
    plsc.subcore_barrier()

    pltpu.sync_copy(dst_hbm.at[w], dst_v)

    def body(j, _):
        pltpu.sync_copy(ones_v, acc_sh.at[dst_v.at[j]], add=True)
        return 0

    lax.fori_loop(0, NCH_DEG, body, 0)
    plsc.subcore_barrier()

    pltpu.sync_copy(acc_sh.at[pl.ds(s * ROWS_PER_TILE, ROWS_PER_TILE)], zb_v)
    pltpu.sync_copy(
        zb_v, deg_out.at[pl.ds(c * N_ACC + s * ROWS_PER_TILE, ROWS_PER_TILE)]
    )


@functools.partial(
    pl.kernel,
    out_type=jax.ShapeDtypeStruct((NC, N_ACC, D_HID), jnp.float32),
    mesh=_mesh,
    scratch_types=[
        pltpu.VMEM((IB, CHUNK), jnp.int32),
        pltpu.VMEM((IB, CHUNK), jnp.int32),
        pltpu.VMEM((4, CHUNK, D_HID), jnp.float32),
        pltpu.VMEM((2, WROWS, D_HID), jnp.float32),
        pltpu.VMEM_SHARED((N_ACC, D_HID), jnp.float32),
        pltpu.SemaphoreType.DMA,
        pltpu.SemaphoreType.DMA,
        pltpu.SemaphoreType.DMA,
    ],
    compiler_params=pltpu.CompilerParams(use_tc_tiling_on_sc=False),
)
def _scatter_kernel(y_hbm, src_hbm, dst_hbm, acc_out,
                    src_v, dst_v, rows_v, zb_v, acc_sh, gsem, ssem, wsem):
    c = lax.axis_index("c")
    s = lax.axis_index("s")

    def zfill(i, _):
        zb_v[0, i, pl.ds(0, 16)] = jnp.zeros((16,), jnp.float32)
        zb_v[0, i, pl.ds(16, 16)] = jnp.zeros((16,), jnp.float32)
        return 0

    lax.fori_loop(0, WROWS, zfill, 0)
    for z in range(ROWS_PER_TILE // WROWS):
        pltpu.sync_copy(
            zb_v.at[0], acc_sh.at[pl.ds(s * ROWS_PER_TILE + z * WROWS, WROWS)]
        )
    plsc.subcore_barrier()

    def wait_gather(j, b):
        pltpu.make_async_copy(
            y_hbm.at[src_v.at[j]], rows_v.at[b], gsem
        ).wait()

    def wait_scatter():
        pltpu.make_async_copy(
            rows_v.at[0], acc_sh.at[dst_v.at[0]], ssem
        ).wait()

    def blk_body(blk, _):
        pltpu.sync_copy(src_hbm.at[c, s, pl.ds(blk * IB, IB)], src_v)
        pltpu.sync_copy(dst_hbm.at[c, s, pl.ds(blk * IB, IB)], dst_v)

        pltpu.async_copy(y_hbm.at[src_v.at[0]], rows_v.at[0], gsem)
        pltpu.async_copy(y_hbm.at[src_v.at[1]], rows_v.at[1], gsem)

        def body(ii, _):
            for b in range(4):
                j = 4 * ii + b
                @pl.when(j + 2 < IB)
                def _():
                    @pl.when(j >= 2)
                    def _():
                        wait_scatter()

                    pltpu.async_copy(
                        y_hbm.at[src_v.at[j + 2]], rows_v.at[(b + 2) % 4],
                        gsem,
                    )

                wait_gather(j, b)
                pltpu.async_copy(
                    rows_v.at[b], acc_sh.at[dst_v.at[j]], ssem, add=True
                )
            return 0

        lax.fori_loop(0, IB // 4, body, 0)
        for _ in range(4):
            wait_scatter()
        return 0

    lax.fori_loop(0, NBLK, blk_body, 0)
    plsc.subcore_barrier()

    NZ = ROWS_PER_TILE // WROWS

    def wait_writeout(z):
        r0 = s * ROWS_PER_TILE + z * WROWS
        pltpu.make_async_copy(
            zb_v.at[z % 2], acc_out.at[c, pl.ds(r0, WROWS)], wsem
        ).wait()

    for z in range(NZ):
        if z >= 2:
            wait_writeout(z - 2)
        r0 = s * ROWS_PER_TILE + z * WROWS
        pltpu.sync_copy(acc_sh.at[pl.ds(r0, WROWS)], zb_v.at[z % 2])
        pltpu.async_copy(zb_v.at[z % 2], acc_out.at[c, pl.ds(r0, WROWS)], wsem)
    wait_writeout(NZ - 2)
    wait_writeout(NZ - 1)


def _dense1_body(x_ref, deg_ref, w1_ref, y_ref):
    deg = deg_ref[:, 0] + deg_ref[:, 1] + 1.0
    dinv = lax.rsqrt(deg)
    xw = jnp.dot(x_ref[...], w1_ref[...], preferred_element_type=jnp.float32)
    y_ref[...] = xw * dinv[:, None]


def _dense2_body(acc_ref, y_ref, deg_ref, b1_ref, w2_ref, b2_ref, z_ref):
    deg = deg_ref[:, 0] + deg_ref[:, 1] + 1.0
    dinv = lax.rsqrt(deg)
    tot = acc_ref[0] + acc_ref[1] + y_ref[...]
    h = tot * dinv[:, None] + b1_ref[...]
    e = jnp.maximum(h, 0.0)
    z_ref[...] = (
        jnp.dot(e, w2_ref[...], preferred_element_type=jnp.float32)
        + b2_ref[...]
    )


RB = 2000


def kernel(x, edge_index, W1, b1, W2, b2):
    src = edge_index[0].astype(jnp.int32)
    dst = edge_index[1].astype(jnp.int32)
    npad = E_PAD - E
    ar = jnp.arange(npad, dtype=jnp.int32)
    src_p = jnp.concatenate([src, ar % CHUNK])
    dst_p = jnp.concatenate([dst, N + (ar % N_PAD_ROWS)])

    deg = _deg_kernel(dst_p.reshape(NW, NCH_DEG, CHUNK))
    deg = deg.reshape(NC, N_ACC).T

    y = pl.pallas_call(
        _dense1_body,
        grid=(N // RB,),
        in_specs=[
            pl.BlockSpec((RB, D_IN), lambda i: (i, 0)),
            pl.BlockSpec((RB, NC), lambda i: (i, 0)),
            pl.BlockSpec((D_IN, D_HID), lambda i: (0, 0)),
        ],
        out_specs=pl.BlockSpec((RB, D_HID), lambda i: (i, 0)),
        out_shape=jax.ShapeDtypeStruct((N, D_HID), jnp.float32),
    )(x, deg, W1)

    acc = _scatter_kernel(
        y,
        src_p.reshape(NC, NS, NCH_SC, CHUNK),
        dst_p.reshape(NC, NS, NCH_SC, CHUNK),
    )

    z = pl.pallas_call(
        _dense2_body,
        grid=(N // RB,),
        in_specs=[
            pl.BlockSpec((NC, RB, D_HID), lambda i: (0, i, 0)),
            pl.BlockSpec((RB, D_HID), lambda i: (i, 0)),
            pl.BlockSpec((RB, NC), lambda i: (i, 0)),
            pl.BlockSpec((1, D_HID), lambda i: (0, 0)),
            pl.BlockSpec((D_HID, N_CLASS), lambda i: (0, 0)),
            pl.BlockSpec((1, N_CLASS), lambda i: (0, 0)),
        ],
        out_specs=pl.BlockSpec((RB, N_CLASS), lambda i: (i, 0)),
        out_shape=jax.ShapeDtypeStruct((N, N_CLASS), jnp.float32),
    )(acc, y, deg, b1.reshape(1, D_HID), W2, b2.reshape(1, N_CLASS))

    return z


# speedup vs baseline: 67.5752x; 1.1632x over previous
import functools

import jax
import jax.numpy as jnp
from jax import lax
from jax.experimental import pallas as pl
from jax.experimental.pallas import tpu as pltpu
from jax.experimental.pallas import tpu_sc as plsc

N = 50000
E = 800000
D_IN = 64
D_HID = 32
N_CLASS = 40

NC = 2
NS = 16
NW = NC * NS
CHUNK = 128
N_PAD_ROWS = 176
N_ACC = N + N_PAD_ROWS
ROWS_PER_TILE = N_ACC // NS
ZROWS = ROWS_PER_TILE // 8

NCH_DEG = -(-E // (NW * CHUNK))
E_PAD = NW * CHUNK * NCH_DEG
NCH_SC = E_PAD // (NC * NS * CHUNK)
IB = 28
NBLK = NCH_SC // IB
WROWS = ROWS_PER_TILE // 32

_mesh = plsc.VectorSubcoreMesh(core_axis_name="c", subcore_axis_name="s")


@functools.partial(
    pl.kernel,
    out_type=jax.ShapeDtypeStruct((NC * N_ACC,), jnp.float32),
    mesh=_mesh,
    scratch_types=[
        pltpu.VMEM((NCH_DEG, CHUNK), jnp.int32),
        pltpu.VMEM((CHUNK,), jnp.float32),
        pltpu.VMEM((ROWS_PER_TILE,), jnp.float32),
        pltpu.VMEM_SHARED((N_ACC,), jnp.float32),
    ],
)
def _deg_kernel(dst_hbm, deg_out, dst_v, ones_v, zb_v, acc_sh):
    c = lax.axis_index("c")
    s = lax.axis_index("s")
    w = s * NC + c

    def fill(i, _):
        ones_v[pl.ds(i * 16, 16)] = jnp.ones((16,), jnp.float32)
        return 0

    lax.fori_loop(0, CHUNK // 16, fill, 0)

    def zfill(i, _):
        zb_v[pl.ds(i * 16, 16)] = jnp.zeros((16,), jnp.float32)
        return 0

    lax.fori_loop(0, ROWS_PER_TILE // 16, zfill, 0)

    pltpu.sync_copy(zb_v, acc_sh.at[pl.ds(s * ROWS_PER_TILE, ROWS_PER_TILE)])
    plsc.subcore_barrier()

    pltpu.sync_copy(dst_hbm.at[w], dst_v)

    def body(j, _):
        pltpu.sync_copy(ones_v, acc_sh.at[dst_v.at[j]], add=True)
        return 0

    lax.fori_loop(0, NCH_DEG, body, 0)
    plsc.subcore_barrier()

    pltpu.sync_copy(acc_sh.at[pl.ds(s * ROWS_PER_TILE, ROWS_PER_TILE)], zb_v)
    pltpu.sync_copy(
        zb_v, deg_out.at[pl.ds(c * N_ACC + s * ROWS_PER_TILE, ROWS_PER_TILE)]
    )


@functools.partial(
    pl.kernel,
    out_type=jax.ShapeDtypeStruct((NC, N_ACC, D_HID), jnp.float32),
    mesh=_mesh,
    scratch_types=[
        pltpu.VMEM((IB, CHUNK), jnp.int32),
        pltpu.VMEM((IB, CHUNK), jnp.int32),
        pltpu.VMEM((4, CHUNK, D_HID), jnp.float32),
        pltpu.VMEM((2, WROWS, D_HID), jnp.float32),
        pltpu.VMEM_SHARED((N_ACC, D_HID), jnp.float32),
        pltpu.SemaphoreType.DMA,
        pltpu.SemaphoreType.DMA,
        pltpu.SemaphoreType.DMA,
    ],
    compiler_params=pltpu.CompilerParams(use_tc_tiling_on_sc=False),
)
def _scatter_kernel(y_hbm, src_hbm, dst_hbm, acc_out,
                    src_v, dst_v, rows_v, zb_v, acc_sh, gsem, ssem, wsem):
    c = lax.axis_index("c")
    s = lax.axis_index("s")

    def zfill(i, _):
        zb_v[0, i, pl.ds(0, 16)] = jnp.zeros((16,), jnp.float32)
        zb_v[0, i, pl.ds(16, 16)] = jnp.zeros((16,), jnp.float32)
        return 0

    lax.fori_loop(0, WROWS, zfill, 0)
    for z in range(ROWS_PER_TILE // WROWS):
        pltpu.sync_copy(
            zb_v.at[0], acc_sh.at[pl.ds(s * ROWS_PER_TILE + z * WROWS, WROWS)]
        )
    plsc.subcore_barrier()

    def wait_gather(j, b):
        pltpu.make_async_copy(
            y_hbm.at[src_v.at[j]], rows_v.at[b], gsem
        ).wait()

    def wait_scatter():
        pltpu.make_async_copy(
            rows_v.at[0], acc_sh.at[dst_v.at[0]], ssem
        ).wait()

    def blk_body(blk, _):
        pltpu.sync_copy(src_hbm.at[c, s, pl.ds(blk * IB, IB)], src_v)
        pltpu.sync_copy(dst_hbm.at[c, s, pl.ds(blk * IB, IB)], dst_v)

        pltpu.async_copy(y_hbm.at[src_v.at[0]], rows_v.at[0], gsem)
        pltpu.async_copy(y_hbm.at[src_v.at[1]], rows_v.at[1], gsem)

        def body(ii, _):
            for b in range(4):
                j = 4 * ii + b
                @pl.when(j + 2 < IB)
                def _():
                    @pl.when(j >= 2)
                    def _():
                        wait_scatter()

                    pltpu.async_copy(
                        y_hbm.at[src_v.at[j + 2]], rows_v.at[(b + 2) % 4],
                        gsem,
                    )

                wait_gather(j, b)
                pltpu.async_copy(
                    rows_v.at[b], acc_sh.at[dst_v.at[j]], ssem, add=True
                )
            return 0

        lax.fori_loop(0, IB // 4, body, 0)
        for _ in range(4):
            wait_scatter()
        return 0

    lax.fori_loop(0, NBLK, blk_body, 0)
    plsc.subcore_barrier()

    NZ = ROWS_PER_TILE // WROWS

    def wait_writeout(z):
        r0 = s * ROWS_PER_TILE + z * WROWS
        pltpu.make_async_copy(
            zb_v.at[z % 2], acc_out.at[c, pl.ds(r0, WROWS)], wsem
        ).wait()

    for z in range(NZ):
        if z >= 2:
            wait_writeout(z - 2)
        r0 = s * ROWS_PER_TILE + z * WROWS
        pltpu.sync_copy(acc_sh.at[pl.ds(r0, WROWS)], zb_v.at[z % 2])
        pltpu.async_copy(zb_v.at[z % 2], acc_out.at[c, pl.ds(r0, WROWS)], wsem)
    wait_writeout(NZ - 2)
    wait_writeout(NZ - 1)


def _dense1_body(x4_ref, dinv_ref, w4_ref, y_ref):
    xw = jnp.dot(x4_ref[...], w4_ref[...], preferred_element_type=jnp.float32)
    y_ref[...] = xw * dinv_ref[...]


def _dense2_body(acc_ref, y_ref, dinv_ref, b1_ref, w2_ref, b2_ref, z_ref):
    tot = acc_ref[0] + acc_ref[1] + y_ref[...]
    h = tot * dinv_ref[...] + b1_ref[...]
    e = jnp.maximum(h, 0.0)
    z_ref[...] = (
        jnp.dot(e, w2_ref[...], preferred_element_type=jnp.float32)
        + b2_ref[...]
    )


RB = 6272
GSTEPS = N_ACC // RB


def kernel(x, edge_index, W1, b1, W2, b2):
    src = edge_index[0].astype(jnp.int32)
    dst = edge_index[1].astype(jnp.int32)
    npad = E_PAD - E
    ar = jnp.arange(npad, dtype=jnp.int32)
    src_p = jnp.concatenate([src, ar % CHUNK])
    dst_p = jnp.concatenate([dst, N + (ar % N_PAD_ROWS)])

    deg = _deg_kernel(dst_p.reshape(NW, NCH_DEG, CHUNK))

    PK = 128 // D_HID
    QB = RB // PK
    ZW = PK * N_CLASS

    dd = deg.reshape(NC, N_ACC)
    dinv = lax.rsqrt(dd[0] + dd[1] + 1.0)
    dinv128 = jnp.broadcast_to(dinv[:, None], (N_ACC, D_HID)).reshape(
        N_ACC // PK, 128
    )
    x4 = x.reshape(N // PK, PK * D_IN)
    W4 = jnp.zeros((PK, D_IN, PK, D_HID), jnp.float32)
    W2blk = jnp.zeros((PK, D_HID, PK, N_CLASS), jnp.float32)
    for k in range(PK):
        W4 = W4.at[k, :, k, :].set(W1)
        W2blk = W2blk.at[k, :, k, :].set(W2)
    W4 = W4.reshape(PK * D_IN, 128)
    W2blk = W2blk.reshape(128, ZW)
    b1t = jnp.tile(b1, PK).reshape(1, 128)
    b2t = jnp.tile(b2, PK).reshape(1, ZW)

    y128 = pl.pallas_call(
        _dense1_body,
        grid=(GSTEPS,),
        in_specs=[
            pl.BlockSpec((QB, PK * D_IN), lambda i: (i, 0)),
            pl.BlockSpec((QB, 128), lambda i: (i, 0)),
            pl.BlockSpec((PK * D_IN, 128), lambda i: (0, 0)),
        ],
        out_specs=pl.BlockSpec((QB, 128), lambda i: (i, 0)),
        out_shape=jax.ShapeDtypeStruct((N_ACC // PK, 128), jnp.float32),
    )(x4, dinv128, W4)

    acc = _scatter_kernel(
        y128.reshape(N_ACC, D_HID),
        src_p.reshape(NC, NS, NCH_SC, CHUNK),
        dst_p.reshape(NC, NS, NCH_SC, CHUNK),
    )
    acc128 = acc.reshape(NC, N_ACC // PK, 128)

    z128 = pl.pallas_call(
        _dense2_body,
        grid=(GSTEPS,),
        in_specs=[
            pl.BlockSpec((NC, QB, 128), lambda i: (0, i, 0)),
            pl.BlockSpec((QB, 128), lambda i: (i, 0)),
            pl.BlockSpec((QB, 128), lambda i: (i, 0)),
            pl.BlockSpec((1, 128), lambda i: (0, 0)),
            pl.BlockSpec((128, ZW), lambda i: (0, 0)),
            pl.BlockSpec((1, ZW), lambda i: (0, 0)),
        ],
        out_specs=pl.BlockSpec((QB, ZW), lambda i: (i, 0)),
        out_shape=jax.ShapeDtypeStruct((N_ACC // PK, ZW), jnp.float32),
    )(acc128, y128, dinv128, b1t, W2blk, b2t)

    return z128[: N // PK].reshape(N, N_CLASS)


# speedup vs baseline: 74.2964x; 1.0995x over previous
import functools

import jax
import jax.numpy as jnp
from jax import lax
from jax.experimental import pallas as pl
from jax.experimental.pallas import tpu as pltpu
from jax.experimental.pallas import tpu_sc as plsc

N = 50000
E = 800000
D_IN = 64
D_HID = 32
N_CLASS = 40

NC = 2
NS = 16
NW = NC * NS
CHUNK = 128
N_PAD_ROWS = 176
N_ACC = N + N_PAD_ROWS
ROWS_PER_TILE = N_ACC // NS
ZROWS = ROWS_PER_TILE // 8

NCH_DEG = -(-E // (NW * CHUNK))
E_PAD = NW * CHUNK * NCH_DEG
NCH_SC = E_PAD // (NC * NS * CHUNK)
IB = 28
NBLK = NCH_SC // IB
WROWS = ROWS_PER_TILE // 32

_mesh = plsc.VectorSubcoreMesh(core_axis_name="c", subcore_axis_name="s")


@functools.partial(
    pl.kernel,
    out_type=jax.ShapeDtypeStruct((NC * N_ACC,), jnp.float32),
    mesh=_mesh,
    scratch_types=[
        pltpu.VMEM((NCH_DEG * CHUNK,), jnp.int32),
        pltpu.VMEM((CHUNK,), jnp.float32),
        pltpu.VMEM((ROWS_PER_TILE,), jnp.float32),
        pltpu.VMEM_SHARED((N_ACC,), jnp.float32),
    ],
)
def _deg_kernel(dst_hbm, deg_out, dst_v, ones_v, zb_v, acc_sh):
    c = lax.axis_index("c")
    s = lax.axis_index("s")
    w = s * NC + c

    def fill(i, _):
        ones_v[pl.ds(i * 16, 16)] = jnp.ones((16,), jnp.float32)
        return 0

    lax.fori_loop(0, CHUNK // 16, fill, 0)

    def zfill(i, _):
        zb_v[pl.ds(i * 16, 16)] = jnp.zeros((16,), jnp.float32)
        return 0

    lax.fori_loop(0, ROWS_PER_TILE // 16, zfill, 0)

    pltpu.sync_copy(zb_v, acc_sh.at[pl.ds(s * ROWS_PER_TILE, ROWS_PER_TILE)])
    plsc.subcore_barrier()

    pltpu.sync_copy(dst_hbm.at[pl.ds(w * NCH_DEG * CHUNK, NCH_DEG * CHUNK)],
                    dst_v)

    def body(j, _):
        pltpu.sync_copy(
            ones_v, acc_sh.at[dst_v.at[pl.ds(j * CHUNK, CHUNK)]], add=True
        )
        return 0

    lax.fori_loop(0, NCH_DEG, body, 0)
    plsc.subcore_barrier()

    pltpu.sync_copy(acc_sh.at[pl.ds(s * ROWS_PER_TILE, ROWS_PER_TILE)], zb_v)
    pltpu.sync_copy(
        zb_v, deg_out.at[pl.ds(c * N_ACC + s * ROWS_PER_TILE, ROWS_PER_TILE)]
    )


@functools.partial(
    pl.kernel,
    out_type=jax.ShapeDtypeStruct((NC, N_ACC, D_HID), jnp.float32),
    mesh=_mesh,
    scratch_types=[
        pltpu.VMEM((IB * CHUNK,), jnp.int32),
        pltpu.VMEM((IB * CHUNK,), jnp.int32),
        pltpu.VMEM((4, CHUNK, D_HID), jnp.float32),
        pltpu.VMEM((2, WROWS, D_HID), jnp.float32),
        pltpu.VMEM_SHARED((N_ACC, D_HID), jnp.float32),
        pltpu.SemaphoreType.DMA,
        pltpu.SemaphoreType.DMA,
        pltpu.SemaphoreType.DMA,
    ],
    compiler_params=pltpu.CompilerParams(use_tc_tiling_on_sc=False),
)
def _scatter_kernel(y_hbm, src_hbm, dst_hbm, acc_out,
                    src_v, dst_v, rows_v, zb_v, acc_sh, gsem, ssem, wsem):
    c = lax.axis_index("c")
    s = lax.axis_index("s")

    def zfill(i, _):
        zb_v[0, i, pl.ds(0, 16)] = jnp.zeros((16,), jnp.float32)
        zb_v[0, i, pl.ds(16, 16)] = jnp.zeros((16,), jnp.float32)
        return 0

    lax.fori_loop(0, WROWS, zfill, 0)
    for z in range(ROWS_PER_TILE // WROWS):
        pltpu.sync_copy(
            zb_v.at[0], acc_sh.at[pl.ds(s * ROWS_PER_TILE + z * WROWS, WROWS)]
        )
    plsc.subcore_barrier()

    def idx(v, j):
        return v.at[pl.ds(j * CHUNK, CHUNK)]

    def wait_gather(j, b):
        pltpu.make_async_copy(
            y_hbm.at[idx(src_v, j)], rows_v.at[b], gsem
        ).wait()

    def wait_scatter():
        pltpu.make_async_copy(
            rows_v.at[0], acc_sh.at[idx(dst_v, 0)], ssem
        ).wait()

    def blk_body(blk, _):
        off = ((c * NS + s) * NCH_SC + blk * IB) * CHUNK
        pltpu.sync_copy(src_hbm.at[pl.ds(off, IB * CHUNK)], src_v)
        pltpu.sync_copy(dst_hbm.at[pl.ds(off, IB * CHUNK)], dst_v)

        pltpu.async_copy(y_hbm.at[idx(src_v, 0)], rows_v.at[0], gsem)
        pltpu.async_copy(y_hbm.at[idx(src_v, 1)], rows_v.at[1], gsem)

        def body(ii, _):
            for b in range(4):
                j = 4 * ii + b
                @pl.when(j + 2 < IB)
                def _():
                    @pl.when(j >= 2)
                    def _():
                        wait_scatter()

                    pltpu.async_copy(
                        y_hbm.at[idx(src_v, j + 2)], rows_v.at[(b + 2) % 4],
                        gsem,
                    )

                wait_gather(j, b)
                pltpu.async_copy(
                    rows_v.at[b], acc_sh.at[idx(dst_v, j)], ssem, add=True
                )
            return 0

        lax.fori_loop(0, IB // 4, body, 0)
        for _ in range(4):
            wait_scatter()
        return 0

    lax.fori_loop(0, NBLK, blk_body, 0)
    plsc.subcore_barrier()

    NZ = ROWS_PER_TILE // WROWS

    def wait_writeout(z):
        r0 = s * ROWS_PER_TILE + z * WROWS
        pltpu.make_async_copy(
            zb_v.at[z % 2], acc_out.at[c, pl.ds(r0, WROWS)], wsem
        ).wait()

    for z in range(NZ):
        if z >= 2:
            wait_writeout(z - 2)
        r0 = s * ROWS_PER_TILE + z * WROWS
        pltpu.sync_copy(acc_sh.at[pl.ds(r0, WROWS)], zb_v.at[z % 2])
        pltpu.async_copy(zb_v.at[z % 2], acc_out.at[c, pl.ds(r0, WROWS)], wsem)
    wait_writeout(NZ - 2)
    wait_writeout(NZ - 1)


def _dense1_body(x4_ref, dinv_ref, w4_ref, y_ref):
    xw = jnp.dot(x4_ref[...], w4_ref[...], preferred_element_type=jnp.float32)
    y_ref[...] = xw * dinv_ref[...]


def _dense2_body(acc_ref, y_ref, dinv_ref, b1_ref, w2_ref, b2_ref, z_ref):
    tot = acc_ref[0] + acc_ref[1] + y_ref[...]
    h = tot * dinv_ref[...] + b1_ref[...]
    e = jnp.maximum(h, 0.0)
    qb = e.shape[0]
    for k in range(128 // D_HID):
        ek = e[:, k * D_HID:(k + 1) * D_HID]
        zk = (
            jnp.dot(ek, w2_ref[...], preferred_element_type=jnp.float32)
            + b2_ref[...]
        )
        z_ref[pl.Slice(k, qb, 128 // D_HID), :] = zk


RB = 6272
GSTEPS = N_ACC // RB


def kernel(x, edge_index, W1, b1, W2, b2):
    src = edge_index[0].astype(jnp.int32)
    dst = edge_index[1].astype(jnp.int32)
    npad = E_PAD - E
    ar = jnp.arange(npad, dtype=jnp.int32)
    src_p = jnp.concatenate([src, ar % CHUNK])
    dst_p = jnp.concatenate([dst, N + (ar % N_PAD_ROWS)])

    deg = _deg_kernel(dst_p)

    PK = 128 // D_HID
    QB = RB // PK
    ZW = PK * N_CLASS

    dd = deg.reshape(NC, N_ACC)
    dinv = lax.rsqrt(dd[0] + dd[1] + 1.0)
    dinv128 = jnp.broadcast_to(dinv[:, None], (N_ACC, D_HID)).reshape(
        N_ACC // PK, 128
    )
    x4 = x.reshape(N // PK, PK * D_IN)
    W4 = jnp.zeros((PK, D_IN, PK, D_HID), jnp.float32)
    for k in range(PK):
        W4 = W4.at[k, :, k, :].set(W1)
    W4 = W4.reshape(PK * D_IN, 128)
    b1t = jnp.tile(b1, PK).reshape(1, 128)

    y128 = pl.pallas_call(
        _dense1_body,
        grid=(GSTEPS,),
        in_specs=[
            pl.BlockSpec((QB, PK * D_IN), lambda i: (i, 0)),
            pl.BlockSpec((QB, 128), lambda i: (i, 0)),
            pl.BlockSpec((PK * D_IN, 128), lambda i: (0, 0)),
        ],
        out_specs=pl.BlockSpec((QB, 128), lambda i: (i, 0)),
        out_shape=jax.ShapeDtypeStruct((N_ACC // PK, 128), jnp.float32),
    )(x4, dinv128, W4)

    acc = _scatter_kernel(y128.reshape(N_ACC, D_HID), src_p, dst_p)
    acc128 = acc.reshape(NC, N_ACC // PK, 128)

    z = pl.pallas_call(
        _dense2_body,
        grid=(GSTEPS,),
        in_specs=[
            pl.BlockSpec((NC, QB, 128), lambda i: (0, i, 0)),
            pl.BlockSpec((QB, 128), lambda i: (i, 0)),
            pl.BlockSpec((QB, 128), lambda i: (i, 0)),
            pl.BlockSpec((1, 128), lambda i: (0, 0)),
            pl.BlockSpec((D_HID, N_CLASS), lambda i: (0, 0)),
            pl.BlockSpec((1, N_CLASS), lambda i: (0, 0)),
        ],
        out_specs=pl.BlockSpec((RB, N_CLASS), lambda i: (i, 0)),
        out_shape=jax.ShapeDtypeStruct((N, N_CLASS), jnp.float32),
    )(acc128, y128, dinv128, b1t, W2, b2.reshape(1, N_CLASS))

    return z


# speedup vs baseline: 76.1186x; 1.0245x over previous
import functools

import jax
import jax.numpy as jnp
from jax import lax
from jax.experimental import pallas as pl
from jax.experimental.pallas import tpu as pltpu
from jax.experimental.pallas import tpu_sc as plsc

N = 50000
E = 800000
D_IN = 64
D_HID = 32
N_CLASS = 40

NC = 2
NS = 16
NW = NC * NS
CHUNK = 128
N_PAD_ROWS = 176
N_ACC = N + N_PAD_ROWS
ROWS_PER_TILE = N_ACC // NS
ZROWS = ROWS_PER_TILE // 8

NCH_DEG = -(-E // (NW * CHUNK))
E_PAD = NW * CHUNK * NCH_DEG
NCH_SC = E_PAD // (NC * NS * CHUNK)
IB = 28
NBLK = NCH_SC // IB
WROWS = ROWS_PER_TILE // 32

_mesh = plsc.VectorSubcoreMesh(core_axis_name="c", subcore_axis_name="s")


@functools.partial(
    pl.kernel,
    out_type=jax.ShapeDtypeStruct((NC * N_ACC,), jnp.float32),
    mesh=_mesh,
    scratch_types=[
        pltpu.VMEM((NCH_DEG * CHUNK,), jnp.int32),
        pltpu.VMEM((CHUNK,), jnp.float32),
        pltpu.VMEM((ROWS_PER_TILE,), jnp.float32),
        pltpu.VMEM_SHARED((N_ACC,), jnp.float32),
    ],
)
def _deg_kernel(dst_hbm, deg_out, dst_v, ones_v, zb_v, acc_sh):
    c = lax.axis_index("c")
    s = lax.axis_index("s")
    w = s * NC + c

    def fill(i, _):
        ones_v[pl.ds(i * 16, 16)] = jnp.ones((16,), jnp.float32)
        return 0

    lax.fori_loop(0, CHUNK // 16, fill, 0)

    def zfill(i, _):
        zb_v[pl.ds(i * 16, 16)] = jnp.zeros((16,), jnp.float32)
        return 0

    lax.fori_loop(0, ROWS_PER_TILE // 16, zfill, 0)

    pltpu.sync_copy(zb_v, acc_sh.at[pl.ds(s * ROWS_PER_TILE, ROWS_PER_TILE)])
    plsc.subcore_barrier()

    pltpu.sync_copy(dst_hbm.at[pl.ds(w * NCH_DEG * CHUNK, NCH_DEG * CHUNK)],
                    dst_v)

    def body(j, _):
        pltpu.sync_copy(
            ones_v, acc_sh.at[dst_v.at[pl.ds(j * CHUNK, CHUNK)]], add=True
        )
        return 0

    lax.fori_loop(0, NCH_DEG, body, 0)
    plsc.subcore_barrier()

    pltpu.sync_copy(acc_sh.at[pl.ds(s * ROWS_PER_TILE, ROWS_PER_TILE)], zb_v)
    pltpu.sync_copy(
        zb_v, deg_out.at[pl.ds(c * N_ACC + s * ROWS_PER_TILE, ROWS_PER_TILE)]
    )


@functools.partial(
    pl.kernel,
    out_type=jax.ShapeDtypeStruct((NC, N_ACC, D_HID), jnp.float32),
    mesh=_mesh,
    scratch_types=[
        pltpu.VMEM((IB * CHUNK,), jnp.int32),
        pltpu.VMEM((IB * CHUNK,), jnp.int32),
        pltpu.VMEM((4, CHUNK, D_HID), jnp.float32),
        pltpu.VMEM((2, WROWS, D_HID), jnp.float32),
        pltpu.VMEM_SHARED((N_ACC, D_HID), jnp.float32),
        pltpu.SemaphoreType.DMA,
        pltpu.SemaphoreType.DMA,
        pltpu.SemaphoreType.DMA,
    ],
    compiler_params=pltpu.CompilerParams(use_tc_tiling_on_sc=False),
)
def _scatter_kernel(y_hbm, src_hbm, dst_hbm, acc_out,
                    src_v, dst_v, rows_v, zb_v, acc_sh, gsem, ssem, wsem):
    c = lax.axis_index("c")
    s = lax.axis_index("s")

    def zfill(i, _):
        zb_v[0, i, pl.ds(0, 16)] = jnp.zeros((16,), jnp.float32)
        zb_v[0, i, pl.ds(16, 16)] = jnp.zeros((16,), jnp.float32)
        return 0

    lax.fori_loop(0, WROWS, zfill, 0)
    for z in range(ROWS_PER_TILE // WROWS):
        pltpu.sync_copy(
            zb_v.at[0], acc_sh.at[pl.ds(s * ROWS_PER_TILE + z * WROWS, WROWS)]
        )
    plsc.subcore_barrier()

    def idx(v, j):
        return v.at[pl.ds(j * CHUNK, CHUNK)]

    def wait_gather(j, b):
        pltpu.make_async_copy(
            y_hbm.at[idx(src_v, j)], rows_v.at[b], gsem
        ).wait()

    def wait_scatter():
        pltpu.make_async_copy(
            rows_v.at[0], acc_sh.at[idx(dst_v, 0)], ssem
        ).wait()

    def blk_body(blk, _):
        off = ((c * NS + s) * NCH_SC + blk * IB) * CHUNK
        pltpu.sync_copy(src_hbm.at[pl.ds(off, IB * CHUNK)], src_v)
        pltpu.sync_copy(dst_hbm.at[pl.ds(off, IB * CHUNK)], dst_v)

        pltpu.async_copy(y_hbm.at[idx(src_v, 0)], rows_v.at[0], gsem)
        pltpu.async_copy(y_hbm.at[idx(src_v, 1)], rows_v.at[1], gsem)

        def body(ii, _):
            for b in range(4):
                j = 4 * ii + b
                @pl.when(j + 2 < IB)
                def _():
                    @pl.when(j >= 2)
                    def _():
                        wait_scatter()

                    pltpu.async_copy(
                        y_hbm.at[idx(src_v, j + 2)], rows_v.at[(b + 2) % 4],
                        gsem,
                    )

                wait_gather(j, b)
                pltpu.async_copy(
                    rows_v.at[b], acc_sh.at[idx(dst_v, j)], ssem, add=True
                )
            return 0

        lax.fori_loop(0, IB // 4, body, 0)
        for _ in range(4):
            wait_scatter()
        return 0

    lax.fori_loop(0, NBLK, blk_body, 0)
    plsc.subcore_barrier()

    NZ = ROWS_PER_TILE // WROWS

    def wait_writeout(z):
        r0 = s * ROWS_PER_TILE + z * WROWS
        pltpu.make_async_copy(
            zb_v.at[z % 2], acc_out.at[c, pl.ds(r0, WROWS)], wsem
        ).wait()

    for z in range(NZ):
        if z >= 2:
            wait_writeout(z - 2)
        r0 = s * ROWS_PER_TILE + z * WROWS
        pltpu.sync_copy(acc_sh.at[pl.ds(r0, WROWS)], zb_v.at[z % 2])
        pltpu.async_copy(zb_v.at[z % 2], acc_out.at[c, pl.ds(r0, WROWS)], wsem)
    wait_writeout(NZ - 2)
    wait_writeout(NZ - 1)


def _dinv128(dsum4_ref, e_ref):
    d = jnp.dot(dsum4_ref[...], e_ref[...], preferred_element_type=jnp.float32)
    return lax.rsqrt(d)


def _dense1_body(x4_ref, dsum4_ref, e_ref, w4_ref, y_ref):
    xw = jnp.dot(x4_ref[...], w4_ref[...], preferred_element_type=jnp.float32)
    y_ref[...] = xw * _dinv128(dsum4_ref, e_ref)


def _dense2_body(acc_ref, y_ref, dsum4_ref, e_ref, b1_ref, w2_ref, b2_ref,
                 z_ref):
    tot = acc_ref[0] + acc_ref[1] + y_ref[...]
    h = tot * _dinv128(dsum4_ref, e_ref) + b1_ref[...]
    e = jnp.maximum(h, 0.0)
    qb = e.shape[0]
    for k in range(128 // D_HID):
        ek = e[:, k * D_HID:(k + 1) * D_HID]
        zk = (
            jnp.dot(ek, w2_ref[...], preferred_element_type=jnp.float32)
            + b2_ref[...]
        )
        z_ref[pl.Slice(k, qb, 128 // D_HID), :] = zk


RB = 6272
GSTEPS = N_ACC // RB


def kernel(x, edge_index, W1, b1, W2, b2):
    npad = E_PAD - E
    ar = jnp.arange(npad, dtype=jnp.int32)
    dst = edge_index[1].astype(jnp.int32)
    dst_p = jnp.concatenate([dst, N + (ar % N_PAD_ROWS)])
    dst_p = lax.optimization_barrier(dst_p)
    deg = _deg_kernel(dst_p)
    src = edge_index[0].astype(jnp.int32)
    src_p = jnp.concatenate([src, ar % CHUNK])

    PK = 128 // D_HID
    QB = RB // PK
    ZW = PK * N_CLASS

    dsum4 = (deg[:N_ACC] + deg[N_ACC:] + 1.0).reshape(N_ACC // PK, PK)
    rep = jnp.repeat(jnp.eye(PK, dtype=jnp.float32), D_HID, axis=1)
    x4 = x.reshape(N // PK, PK * D_IN)
    W4 = jnp.zeros((PK, D_IN, PK, D_HID), jnp.float32)
    for k in range(PK):
        W4 = W4.at[k, :, k, :].set(W1)
    W4 = W4.reshape(PK * D_IN, 128)
    b1t = jnp.tile(b1, PK).reshape(1, 128)

    y128 = pl.pallas_call(
        _dense1_body,
        grid=(GSTEPS,),
        in_specs=[
            pl.BlockSpec((QB, PK * D_IN), lambda i: (i, 0)),
            pl.BlockSpec((QB, PK), lambda i: (i, 0)),
            pl.BlockSpec((PK, 128), lambda i: (0, 0)),
            pl.BlockSpec((PK * D_IN, 128), lambda i: (0, 0)),
        ],
        out_specs=pl.BlockSpec((QB, 128), lambda i: (i, 0)),
        out_shape=jax.ShapeDtypeStruct((N_ACC // PK, 128), jnp.float32),
    )(x4, dsum4, rep, W4)

    acc = _scatter_kernel(y128.reshape(N_ACC, D_HID), src_p, dst_p)
    acc128 = acc.reshape(NC, N_ACC // PK, 128)

    z = pl.pallas_call(
        _dense2_body,
        grid=(GSTEPS,),
        in_specs=[
            pl.BlockSpec((NC, QB, 128), lambda i: (0, i, 0)),
            pl.BlockSpec((QB, 128), lambda i: (i, 0)),
            pl.BlockSpec((QB, PK), lambda i: (i, 0)),
            pl.BlockSpec((PK, 128), lambda i: (0, 0)),
            pl.BlockSpec((1, 128), lambda i: (0, 0)),
            pl.BlockSpec((D_HID, N_CLASS), lambda i: (0, 0)),
            pl.BlockSpec((1, N_CLASS), lambda i: (0, 0)),
        ],
        out_specs=pl.BlockSpec((RB, N_CLASS), lambda i: (i, 0)),
        out_shape=jax.ShapeDtypeStruct((N, N_CLASS), jnp.float32),
    )(acc128, y128, dsum4, rep, b1t, W2, b2.reshape(1, N_CLASS))

    return z
